# baseline (speedup 1.0000x reference)
import jax
import jax.numpy as jnp
from jax.experimental import pallas as pl
from jax.experimental.pallas import tpu as pltpu


def _make_conv_stack_kernel(n_hidden, H, W, C, B):
    WW = W + 8
    Hh, Wh = H // 2, W // 2
    M = H * WW
    P3R = (H + 2) * WW

    def body(*refs):
        x_ref = refs[0]
        w1_ref, b1_ref = refs[1], refs[2]
        wb = refs[3:3 + 2 * n_hidden]
        o_ref = refs[3 + 2 * n_hidden]
        scr = refs[3 + 2 * n_hidden + 1:]
        p3s, obufs, pools = scr[0:B], scr[B:2 * B], scr[2 * B:3 * B]

        zb = jnp.zeros((48, 3 * C), jnp.bfloat16)
        for p3 in p3s:
            p3[pl.ds(0, 48), :] = zb
            p3[pl.ds(P3R - 48, 48), :] = zb

        bidx = jax.lax.broadcasted_iota(jnp.int32, (H, WW, C), 1)
        valid = (bidx < W).reshape(M, C)

        def store_shifted(p3, y):
            yb = jnp.where(valid, y, 0.0).astype(jnp.bfloat16)
            p3[pl.ds(WW + 1, M), 0:C] = yb
            p3[pl.ds(WW, M), C:2 * C] = yb
            p3[pl.ds(WW - 1, M), 2 * C:3 * C] = yb

        for b in range(B):
            y = jax.lax.dot_general(
                x_ref[b].reshape(M, 9), w1_ref[...],
                dimension_numbers=(((1,), (0,)), ((), ())),
                preferred_element_type=jnp.float32)
            store_shifted(p3s[b], jnp.maximum(y + b1_ref[...], 0.0))

        ys = [None] * B
        for l in range(n_hidden):
            w_ref, b_ref = wb[2 * l], wb[2 * l + 1]
            for b in range(B):
                obufs[b][...] = jax.lax.dot_general(
                    p3s[b][...], w_ref[...],
                    dimension_numbers=(((1,), (0,)), ((), ())),
                    preferred_element_type=jnp.float32)
            for b in range(B):
                obuf = obufs[b]
                y = (obuf[pl.ds(0, M), 0:C]
                     + obuf[pl.ds(WW, M), C:2 * C]
                     + obuf[pl.ds(2 * WW, M), 2 * C:3 * C])
                y = jnp.maximum(y + b_ref[...], 0.0)
                if l != n_hidden - 1:
                    store_shifted(p3s[b], y)
                else:
                    ys[b] = y

        for b in range(B):
            yp = ys[b].reshape(Hh, 2, WW, C)
            pool_buf = pools[b]
            pool_buf[...] = jnp.maximum(yp[:, 0], yp[:, 1])
            p = jnp.maximum(pool_buf[:, pl.ds(0, Wh, 2), :],
                            pool_buf[:, pl.ds(1, Wh, 2), :])
            o_ref[b] = p.astype(o_ref.dtype)

    return body


def _conv_stack(x9, conv1_w, conv1_b, convs_w, convs_b):
    N, H, WW, _ = x9.shape
    W = WW - 8
    C = conv1_w.shape[1]
    Hh, Wh = H // 2, W // 2
    n_hidden = len(convs_w)
    B = 2 if N % 2 == 0 else 1

    in_specs = [pl.BlockSpec((B, H, WW, 9), lambda n: (n, 0, 0, 0)),
                pl.BlockSpec((9, C), lambda n: (0, 0)),
                pl.BlockSpec((1, C), lambda n: (0, 0))]
    args = [x9, conv1_w, conv1_b]
    for w, b in zip(convs_w, convs_b):
        in_specs.append(pl.BlockSpec((3 * C, 3 * C), lambda n: (0, 0)))
        in_specs.append(pl.BlockSpec((1, C), lambda n: (0, 0)))
        args.append(w)
        args.append(b)

    body = _make_conv_stack_kernel(n_hidden, H, W, C, B)
    return pl.pallas_call(
        body,
        grid=(N // B,),
        out_shape=jax.ShapeDtypeStruct((N, Hh, Wh, C), jnp.bfloat16),
        in_specs=in_specs,
        out_specs=pl.BlockSpec((B, Hh, Wh, C), lambda n: (n, 0, 0, 0)),
        scratch_shapes=(
            [pltpu.VMEM(((H + 2) * WW, 3 * C), jnp.bfloat16) for _ in range(B)]
            + [pltpu.VMEM(((H + 2) * WW, 3 * C), jnp.float32) for _ in range(B)]
            + [pltpu.VMEM((Hh, WW, C), jnp.float32) for _ in range(B)]),
        compiler_params=pltpu.CompilerParams(
            dimension_semantics=("parallel",),
            vmem_limit_bytes=64 * 1024 * 1024),
    )(*args)


def _make_dense_head_kernel(nk):
    def body(x_ref, w1_ref, b1_ref, w2_ref, b2_ref, o_ref, acc_ref):
        k = pl.program_id(1)
        part = jax.lax.dot_general(
            x_ref[...], w1_ref[...].astype(jnp.bfloat16),
            dimension_numbers=(((1,), (0,)), ((), ())),
            preferred_element_type=jnp.float32)
        @pl.when(k == 0)
        def _():
            acc_ref[...] = part

        @pl.when(k > 0)
        def _():
            acc_ref[...] += part

        @pl.when(k == nk - 1)
        def _():
            h = jnp.maximum(acc_ref[...] + b1_ref[...], 0.0)
            o_ref[...] = (jax.lax.dot_general(
                h, w2_ref[...],
                dimension_numbers=(((1,), (0,)), ((), ())),
                preferred_element_type=jnp.float32) + b2_ref[...]
            ).astype(o_ref.dtype)

    return body


def _dense_head(x_flat, fc1_w_t, fc1_b, fc2_w_t, fc2_b):
    N, F = x_flat.shape
    hidden = fc1_w_t.shape[1]
    ncls = fc2_w_t.shape[1]
    nb = 2 if N % 2 == 0 else 1
    Mb = N // nb
    nk, Fb = 8, F // 8
    return pl.pallas_call(
        _make_dense_head_kernel(nk),
        grid=(nb, nk),
        out_shape=jax.ShapeDtypeStruct((N, ncls), jnp.float32),
        in_specs=[
            pl.BlockSpec((Mb, Fb), lambda n, k: (n, k)),
            pl.BlockSpec((Fb, hidden), lambda n, k: (k, 0)),
            pl.BlockSpec((1, hidden), lambda n, k: (0, 0)),
            pl.BlockSpec((hidden, ncls), lambda n, k: (0, 0)),
            pl.BlockSpec((1, ncls), lambda n, k: (0, 0)),
        ],
        out_specs=pl.BlockSpec((Mb, ncls), lambda n, k: (n, 0)),
        scratch_shapes=[pltpu.VMEM((Mb, hidden), jnp.float32)],
        compiler_params=pltpu.CompilerParams(
            dimension_semantics=("parallel", "arbitrary"),
            vmem_limit_bytes=64 * 1024 * 1024),
    )(x_flat, fc1_w_t, fc1_b, fc2_w_t, fc2_b)


def kernel(x, conv1_w, conv1_b,
           convs_w_0, convs_b_0, convs_w_1, convs_b_1,
           convs_w_2, convs_b_2, convs_w_3, convs_b_3,
           convs_w_4, convs_b_4,
           fc1_w_t, fc1_b, fc2_w_t, fc2_b):
    N, _, H, W = x.shape
    C = conv1_w.shape[1]

    xs = x[:, 0, :, :]
    xp = jnp.pad(xs, ((0, 0), (1, 1), (1, 1 + 8)))
    x9 = jnp.stack([xp[:, dy:dy + H, dx:dx + W + 8]
                    for dy in range(3) for dx in range(3)],
                   axis=-1).astype(jnp.bfloat16)

    convs_w = [convs_w_0, convs_w_1, convs_w_2, convs_w_3, convs_w_4]
    convs_b = [convs_b_0, convs_b_1, convs_b_2, convs_b_3, convs_b_4]
    convs_w = [w.reshape(3, 3, C, C).transpose(1, 2, 0, 3).reshape(3 * C, 3 * C)
               .astype(jnp.bfloat16) for w in convs_w]

    feat = _conv_stack(x9, conv1_w.astype(jnp.bfloat16), conv1_b,
                       convs_w, convs_b)
    x_flat = feat.reshape(N, (H // 2) * (W // 2) * C)
    return _dense_head(x_flat, fc1_w_t, fc1_b, fc2_w_t, fc2_b)


# speedup vs baseline: 1.0458x; 1.0458x over previous
import jax
import jax.numpy as jnp
from jax.experimental import pallas as pl
from jax.experimental.pallas import tpu as pltpu


def _make_conv_stack_kernel(n_hidden, H, W, C, B):
    WW = W + 8
    Hh, Wh = H // 2, W // 2
    M = H * WW
    P3R = (H + 2) * WW

    def body(*refs):
        x_ref = refs[0]
        w1_ref, b1_ref = refs[1], refs[2]
        wb = refs[3:3 + 2 * n_hidden]
        o_ref = refs[3 + 2 * n_hidden]
        p3, obuf, pool_buf = refs[3 + 2 * n_hidden + 1:]

        zb = jnp.zeros((48, 3 * C), jnp.bfloat16)
        p3[pl.ds(0, 48), :] = zb
        p3[pl.ds(P3R - 48, 48), :] = zb

        bidx = jax.lax.broadcasted_iota(jnp.int32, (H, WW, C), 1)
        valid = (bidx < W).reshape(M, C)

        def store_shifted(y):
            yb = jnp.where(valid, y, 0.0).astype(jnp.bfloat16)
            p3[pl.ds(WW + 1, M), 0:C] = yb
            p3[pl.ds(WW, M), C:2 * C] = yb
            p3[pl.ds(WW - 1, M), 2 * C:3 * C] = yb

        for b in range(B):
            y = jax.lax.dot_general(
                x_ref[b].reshape(M, 9), w1_ref[...],
                dimension_numbers=(((1,), (0,)), ((), ())),
                preferred_element_type=jnp.float32)
            store_shifted(jnp.maximum(y + b1_ref[...], 0.0))

            for l in range(n_hidden):
                w_ref, b_ref = wb[2 * l], wb[2 * l + 1]
                obuf[...] = jax.lax.dot_general(
                    p3[...], w_ref[...],
                    dimension_numbers=(((1,), (0,)), ((), ())),
                    preferred_element_type=jnp.float32)
                y = (obuf[pl.ds(0, M), 0:C]
                     + obuf[pl.ds(WW, M), C:2 * C]
                     + obuf[pl.ds(2 * WW, M), 2 * C:3 * C])
                y = jnp.maximum(y + b_ref[...], 0.0)
                if l != n_hidden - 1:
                    store_shifted(y)

            yp = y.reshape(Hh, 2, WW, C)
            pool_buf[...] = jnp.maximum(yp[:, 0], yp[:, 1])
            p = jnp.maximum(pool_buf[:, pl.ds(0, Wh, 2), :],
                            pool_buf[:, pl.ds(1, Wh, 2), :])
            o_ref[b] = p.astype(o_ref.dtype)

    return body


def _conv_stack(x9, conv1_w, conv1_b, convs_w, convs_b):
    N, H, WW, _ = x9.shape
    W = WW - 8
    C = conv1_w.shape[1]
    Hh, Wh = H // 2, W // 2
    n_hidden = len(convs_w)
    B = 2 if N % 2 == 0 else 1

    in_specs = [pl.BlockSpec((B, H, WW, 9), lambda n: (n, 0, 0, 0)),
                pl.BlockSpec((9, C), lambda n: (0, 0)),
                pl.BlockSpec((1, C), lambda n: (0, 0))]
    args = [x9, conv1_w, conv1_b]
    for w, b in zip(convs_w, convs_b):
        in_specs.append(pl.BlockSpec((3 * C, 3 * C), lambda n: (0, 0)))
        in_specs.append(pl.BlockSpec((1, C), lambda n: (0, 0)))
        args.append(w)
        args.append(b)

    body = _make_conv_stack_kernel(n_hidden, H, W, C, B)
    return pl.pallas_call(
        body,
        grid=(N // B,),
        out_shape=jax.ShapeDtypeStruct((N, Hh, Wh, C), jnp.bfloat16),
        in_specs=in_specs,
        out_specs=pl.BlockSpec((B, Hh, Wh, C), lambda n: (n, 0, 0, 0)),
        scratch_shapes=[pltpu.VMEM(((H + 2) * WW, 3 * C), jnp.bfloat16),
                        pltpu.VMEM(((H + 2) * WW, 3 * C), jnp.float32),
                        pltpu.VMEM((Hh, WW, C), jnp.float32)],
        compiler_params=pltpu.CompilerParams(
            dimension_semantics=("parallel",),
            vmem_limit_bytes=64 * 1024 * 1024),
    )(*args)


def _make_dense_head_kernel(nk):
    def body(x_ref, w1_ref, b1_ref, w2_ref, b2_ref, o_ref, acc_ref):
        k = pl.program_id(1)
        part = jax.lax.dot_general(
            x_ref[...], w1_ref[...].astype(jnp.bfloat16),
            dimension_numbers=(((1,), (0,)), ((), ())),
            preferred_element_type=jnp.float32)
        @pl.when(k == 0)
        def _():
            acc_ref[...] = part

        @pl.when(k > 0)
        def _():
            acc_ref[...] += part

        @pl.when(k == nk - 1)
        def _():
            h = jnp.maximum(acc_ref[...] + b1_ref[...], 0.0)
            o_ref[...] = (jax.lax.dot_general(
                h, w2_ref[...],
                dimension_numbers=(((1,), (0,)), ((), ())),
                preferred_element_type=jnp.float32) + b2_ref[...]
            ).astype(o_ref.dtype)

    return body


def _dense_head(x_flat, fc1_w_t, fc1_b, fc2_w_t, fc2_b):
    N, F = x_flat.shape
    hidden = fc1_w_t.shape[1]
    ncls = fc2_w_t.shape[1]
    nb = 2 if N % 2 == 0 else 1
    Mb = N // nb
    nk, Fb = 8, F // 8
    return pl.pallas_call(
        _make_dense_head_kernel(nk),
        grid=(nb, nk),
        out_shape=jax.ShapeDtypeStruct((N, ncls), jnp.float32),
        in_specs=[
            pl.BlockSpec((Mb, Fb), lambda n, k: (n, k)),
            pl.BlockSpec((Fb, hidden), lambda n, k: (k, 0)),
            pl.BlockSpec((1, hidden), lambda n, k: (0, 0)),
            pl.BlockSpec((hidden, ncls), lambda n, k: (0, 0)),
            pl.BlockSpec((1, ncls), lambda n, k: (0, 0)),
        ],
        out_specs=pl.BlockSpec((Mb, ncls), lambda n, k: (n, 0)),
        scratch_shapes=[pltpu.VMEM((Mb, hidden), jnp.float32)],
        compiler_params=pltpu.CompilerParams(
            dimension_semantics=("parallel", "arbitrary"),
            vmem_limit_bytes=64 * 1024 * 1024),
    )(x_flat, fc1_w_t, fc1_b, fc2_w_t, fc2_b)


def kernel(x, conv1_w, conv1_b,
           convs_w_0, convs_b_0, convs_w_1, convs_b_1,
           convs_w_2, convs_b_2, convs_w_3, convs_b_3,
           convs_w_4, convs_b_4,
           fc1_w_t, fc1_b, fc2_w_t, fc2_b):
    N, _, H, W = x.shape
    C = conv1_w.shape[1]

    xs = x[:, 0, :, :]
    xp = jnp.pad(xs, ((0, 0), (1, 1), (1, 1 + 8)))
    x9 = jnp.stack([xp[:, dy:dy + H, dx:dx + W + 8]
                    for dy in range(3) for dx in range(3)],
                   axis=-1).astype(jnp.bfloat16)

    convs_w = [convs_w_0, convs_w_1, convs_w_2, convs_w_3, convs_w_4]
    convs_b = [convs_b_0, convs_b_1, convs_b_2, convs_b_3, convs_b_4]
    convs_w = [w.reshape(3, 3, C, C).transpose(1, 2, 0, 3).reshape(3 * C, 3 * C)
               .astype(jnp.bfloat16) for w in convs_w]

    feat = _conv_stack(x9, conv1_w.astype(jnp.bfloat16), conv1_b,
                       convs_w, convs_b)
    x_flat = feat.reshape(N, (H // 2) * (W // 2) * C)
    return _dense_head(x_flat, fc1_w_t, fc1_b, fc2_w_t, fc2_b)


# speedup vs baseline: 1.0620x; 1.0155x over previous
import jax
import jax.numpy as jnp
from jax.experimental import pallas as pl
from jax.experimental.pallas import tpu as pltpu


def _make_conv_stack_kernel(n_hidden, H, W, C, B):
    WW = W + 8
    Hh, Wh = H // 2, W // 2
    M = H * WW
    P3R = (H + 2) * WW

    def body(*refs):
        x_ref = refs[0]
        w1_ref, b1_ref = refs[1], refs[2]
        wb = refs[3:3 + 2 * n_hidden]
        o_ref = refs[3 + 2 * n_hidden]
        p3, obuf, pool_buf = refs[3 + 2 * n_hidden + 1:]

        zb = jnp.zeros((48, 3 * C), jnp.bfloat16)
        p3[pl.ds(0, 48), :] = zb
        p3[pl.ds(P3R - 48, 48), :] = zb

        bidx = jax.lax.broadcasted_iota(jnp.int32, (H, WW, C), 1)
        valid = (bidx < W).reshape(M, C)

        def store_shifted(y):
            yb = jnp.where(valid, y, 0.0).astype(jnp.bfloat16)
            p3[pl.ds(WW + 1, M), 0:C] = yb
            p3[pl.ds(WW, M), C:2 * C] = yb
            p3[pl.ds(WW - 1, M), 2 * C:3 * C] = yb

        for b in range(B):
            y = jax.lax.dot_general(
                x_ref[b].reshape(M, 9), w1_ref[...],
                dimension_numbers=(((1,), (0,)), ((), ())),
                preferred_element_type=jnp.float32)
            store_shifted(jnp.maximum(y + b1_ref[...], 0.0))

            for l in range(n_hidden):
                w_ref, b_ref = wb[2 * l], wb[2 * l + 1]
                obuf[...] = jax.lax.dot_general(
                    p3[...], w_ref[...],
                    dimension_numbers=(((1,), (0,)), ((), ())),
                    preferred_element_type=jnp.float32)
                y = (obuf[pl.ds(0, M), 0:C]
                     + obuf[pl.ds(WW, M), C:2 * C]
                     + obuf[pl.ds(2 * WW, M), 2 * C:3 * C])
                y = jnp.maximum(y + b_ref[...], 0.0)
                if l != n_hidden - 1:
                    store_shifted(y)

            yp = y.reshape(Hh, 2, WW, C)
            pool_buf[...] = jnp.maximum(yp[:, 0], yp[:, 1])
            p = jnp.maximum(pool_buf[:, pl.ds(0, Wh, 2), :],
                            pool_buf[:, pl.ds(1, Wh, 2), :])
            o_ref[b] = p.astype(o_ref.dtype)

    return body


def _conv_stack(x9, conv1_w, conv1_b, convs_w, convs_b):
    N, H, WW, _ = x9.shape
    W = WW - 8
    C = conv1_w.shape[1]
    Hh, Wh = H // 2, W // 2
    n_hidden = len(convs_w)
    B = 2 if N % 2 == 0 else 1

    in_specs = [pl.BlockSpec((B, H, WW, 9), lambda n: (n, 0, 0, 0)),
                pl.BlockSpec((9, C), lambda n: (0, 0)),
                pl.BlockSpec((1, C), lambda n: (0, 0))]
    args = [x9, conv1_w, conv1_b]
    for w, b in zip(convs_w, convs_b):
        in_specs.append(pl.BlockSpec((3 * C, 3 * C), lambda n: (0, 0)))
        in_specs.append(pl.BlockSpec((1, C), lambda n: (0, 0)))
        args.append(w)
        args.append(b)

    body = _make_conv_stack_kernel(n_hidden, H, W, C, B)
    return pl.pallas_call(
        body,
        grid=(N // B,),
        out_shape=jax.ShapeDtypeStruct((N, Hh, Wh, C), jnp.bfloat16),
        in_specs=in_specs,
        out_specs=pl.BlockSpec((B, Hh, Wh, C), lambda n: (n, 0, 0, 0)),
        scratch_shapes=[pltpu.VMEM(((H + 2) * WW, 3 * C), jnp.bfloat16),
                        pltpu.VMEM(((H + 2) * WW, 3 * C), jnp.float32),
                        pltpu.VMEM((Hh, WW, C), jnp.float32)],
        compiler_params=pltpu.CompilerParams(
            dimension_semantics=("parallel",),
            vmem_limit_bytes=64 * 1024 * 1024),
    )(*args)


def _make_dense_head_kernel(nk):
    def body(x_ref, w1_ref, b1_ref, w2_ref, b2_ref, o_ref, acc_ref):
        k = pl.program_id(1)
        part = jax.lax.dot_general(
            x_ref[...], w1_ref[...].astype(jnp.bfloat16),
            dimension_numbers=(((1,), (0,)), ((), ())),
            preferred_element_type=jnp.float32)
        @pl.when(k == 0)
        def _():
            acc_ref[...] = part

        @pl.when(k > 0)
        def _():
            acc_ref[...] += part

        @pl.when(k == nk - 1)
        def _():
            h = jnp.maximum(acc_ref[...] + b1_ref[...], 0.0)
            o_ref[...] = (jax.lax.dot_general(
                h, w2_ref[...],
                dimension_numbers=(((1,), (0,)), ((), ())),
                preferred_element_type=jnp.float32) + b2_ref[...]
            ).astype(o_ref.dtype)

    return body


def _dense_head(x_flat, fc1_w_t, fc1_b, fc2_w_t, fc2_b):
    N, F = x_flat.shape
    hidden = fc1_w_t.shape[1]
    ncls = fc2_w_t.shape[1]
    nb = 2 if N % 2 == 0 else 1
    Mb = N // nb
    nk, Fb = 2, F // 2
    return pl.pallas_call(
        _make_dense_head_kernel(nk),
        grid=(nb, nk),
        out_shape=jax.ShapeDtypeStruct((N, ncls), jnp.float32),
        in_specs=[
            pl.BlockSpec((Mb, Fb), lambda n, k: (n, k)),
            pl.BlockSpec((Fb, hidden), lambda n, k: (k, 0)),
            pl.BlockSpec((1, hidden), lambda n, k: (0, 0)),
            pl.BlockSpec((hidden, ncls), lambda n, k: (0, 0)),
            pl.BlockSpec((1, ncls), lambda n, k: (0, 0)),
        ],
        out_specs=pl.BlockSpec((Mb, ncls), lambda n, k: (n, 0)),
        scratch_shapes=[pltpu.VMEM((Mb, hidden), jnp.float32)],
        compiler_params=pltpu.CompilerParams(
            dimension_semantics=("parallel", "arbitrary"),
            vmem_limit_bytes=64 * 1024 * 1024),
    )(x_flat, fc1_w_t, fc1_b, fc2_w_t, fc2_b)


def kernel(x, conv1_w, conv1_b,
           convs_w_0, convs_b_0, convs_w_1, convs_b_1,
           convs_w_2, convs_b_2, convs_w_3, convs_b_3,
           convs_w_4, convs_b_4,
           fc1_w_t, fc1_b, fc2_w_t, fc2_b):
    N, _, H, W = x.shape
    C = conv1_w.shape[1]

    xs = x[:, 0, :, :]
    xp = jnp.pad(xs, ((0, 0), (1, 1), (1, 1 + 8)))
    x9 = jnp.stack([xp[:, dy:dy + H, dx:dx + W + 8]
                    for dy in range(3) for dx in range(3)],
                   axis=-1).astype(jnp.bfloat16)

    convs_w = [convs_w_0, convs_w_1, convs_w_2, convs_w_3, convs_w_4]
    convs_b = [convs_b_0, convs_b_1, convs_b_2, convs_b_3, convs_b_4]
    convs_w = [w.reshape(3, 3, C, C).transpose(1, 2, 0, 3).reshape(3 * C, 3 * C)
               .astype(jnp.bfloat16) for w in convs_w]

    feat = _conv_stack(x9, conv1_w.astype(jnp.bfloat16), conv1_b,
                       convs_w, convs_b)
    x_flat = feat.reshape(N, (H // 2) * (W // 2) * C)
    return _dense_head(x_flat, fc1_w_t, fc1_b, fc2_w_t, fc2_b)


# speedup vs baseline: 1.0793x; 1.0162x over previous
import jax
import jax.numpy as jnp
from jax.experimental import pallas as pl
from jax.experimental.pallas import tpu as pltpu


def _make_conv_stack_kernel(n_hidden, H, W, C, B):
    WW = W + 8
    Hh, Wh = H // 2, W // 2
    M = H * WW
    P3R = (H + 2) * WW

    def body(*refs):
        x_ref = refs[0]
        w1_ref, b1_ref = refs[1], refs[2]
        wb = refs[3:3 + 2 * n_hidden]
        o_ref = refs[3 + 2 * n_hidden]
        p3, obuf, pool_buf = refs[3 + 2 * n_hidden + 1:]

        zb = jnp.zeros((48, 3 * C), jnp.bfloat16)
        p3[pl.ds(0, 48), :] = zb
        p3[pl.ds(P3R - 48, 48), :] = zb

        bidx = jax.lax.broadcasted_iota(jnp.int32, (H, WW, C), 1)
        valid = (bidx < W).reshape(M, C)

        def store_shifted(y):
            yb = jnp.where(valid, y, 0.0).astype(jnp.bfloat16)
            p3[pl.ds(WW + 1, M), 0:C] = yb
            p3[pl.ds(WW, M), C:2 * C] = yb
            p3[pl.ds(WW - 1, M), 2 * C:3 * C] = yb

        for b in range(B):
            y = jax.lax.dot_general(
                x_ref[b].reshape(M, 9), w1_ref[...],
                dimension_numbers=(((1,), (0,)), ((), ())),
                preferred_element_type=jnp.float32)
            store_shifted(jnp.maximum(y + b1_ref[...], 0.0))

            for l in range(n_hidden):
                w_ref, b_ref = wb[2 * l], wb[2 * l + 1]
                obuf[...] = jax.lax.dot_general(
                    p3[...], w_ref[...],
                    dimension_numbers=(((1,), (0,)), ((), ())),
                    preferred_element_type=jnp.float32)
                y = (obuf[pl.ds(0, M), 0:C]
                     + obuf[pl.ds(WW, M), C:2 * C]
                     + obuf[pl.ds(2 * WW, M), 2 * C:3 * C])
                y = jnp.maximum(y + b_ref[...], 0.0)
                if l != n_hidden - 1:
                    store_shifted(y)

            yp = y.reshape(Hh, 2, WW, C)
            pool_buf[...] = jnp.maximum(yp[:, 0], yp[:, 1])
            p = jnp.maximum(pool_buf[:, pl.ds(0, Wh, 2), :],
                            pool_buf[:, pl.ds(1, Wh, 2), :])
            o_ref[b] = p.astype(o_ref.dtype)

    return body


def _conv_stack(x9, conv1_w, conv1_b, convs_w, convs_b):
    N, H, WW, _ = x9.shape
    W = WW - 8
    C = conv1_w.shape[1]
    Hh, Wh = H // 2, W // 2
    n_hidden = len(convs_w)
    B = 2 if N % 2 == 0 else 1

    in_specs = [pl.BlockSpec((B, H, WW, 9), lambda n: (n, 0, 0, 0)),
                pl.BlockSpec((9, C), lambda n: (0, 0)),
                pl.BlockSpec((1, C), lambda n: (0, 0))]
    args = [x9, conv1_w, conv1_b]
    for w, b in zip(convs_w, convs_b):
        in_specs.append(pl.BlockSpec((3 * C, 3 * C), lambda n: (0, 0)))
        in_specs.append(pl.BlockSpec((1, C), lambda n: (0, 0)))
        args.append(w)
        args.append(b)

    body = _make_conv_stack_kernel(n_hidden, H, W, C, B)
    return pl.pallas_call(
        body,
        grid=(N // B,),
        out_shape=jax.ShapeDtypeStruct((N, Hh, Wh, C), jnp.bfloat16),
        in_specs=in_specs,
        out_specs=pl.BlockSpec((B, Hh, Wh, C), lambda n: (n, 0, 0, 0)),
        scratch_shapes=[pltpu.VMEM(((H + 2) * WW, 3 * C), jnp.bfloat16),
                        pltpu.VMEM(((H + 2) * WW, 3 * C), jnp.float32),
                        pltpu.VMEM((Hh, WW, C), jnp.float32)],
        compiler_params=pltpu.CompilerParams(
            dimension_semantics=("parallel",),
            vmem_limit_bytes=64 * 1024 * 1024),
    )(*args)


def _make_dense_head_kernel(nk):
    def body(x_ref, w1_ref, b1_ref, w2_ref, b2_ref, o_ref, acc_ref):
        k = pl.program_id(1)
        part = jax.lax.dot_general(
            x_ref[...], w1_ref[...].astype(jnp.bfloat16),
            dimension_numbers=(((1,), (0,)), ((), ())),
            preferred_element_type=jnp.float32)
        @pl.when(k == 0)
        def _():
            acc_ref[...] = part

        @pl.when(k > 0)
        def _():
            acc_ref[...] += part

        @pl.when(k == nk - 1)
        def _():
            h = jnp.maximum(acc_ref[...] + b1_ref[...], 0.0)
            o_ref[...] = (jax.lax.dot_general(
                h, w2_ref[...],
                dimension_numbers=(((1,), (0,)), ((), ())),
                preferred_element_type=jnp.float32) + b2_ref[...]
            ).astype(o_ref.dtype)

    return body


def _dense_head(x_flat, fc1_w_t, fc1_b, fc2_w_t, fc2_b):
    N, F = x_flat.shape
    hidden = fc1_w_t.shape[1]
    ncls = fc2_w_t.shape[1]
    nb = 1
    Mb = N // nb
    nk, Fb = 2, F // 2
    return pl.pallas_call(
        _make_dense_head_kernel(nk),
        grid=(nb, nk),
        out_shape=jax.ShapeDtypeStruct((N, ncls), jnp.float32),
        in_specs=[
            pl.BlockSpec((Mb, Fb), lambda n, k: (n, k)),
            pl.BlockSpec((Fb, hidden), lambda n, k: (k, 0)),
            pl.BlockSpec((1, hidden), lambda n, k: (0, 0)),
            pl.BlockSpec((hidden, ncls), lambda n, k: (0, 0)),
            pl.BlockSpec((1, ncls), lambda n, k: (0, 0)),
        ],
        out_specs=pl.BlockSpec((Mb, ncls), lambda n, k: (n, 0)),
        scratch_shapes=[pltpu.VMEM((Mb, hidden), jnp.float32)],
        compiler_params=pltpu.CompilerParams(
            dimension_semantics=("parallel", "arbitrary"),
            vmem_limit_bytes=64 * 1024 * 1024),
    )(x_flat, fc1_w_t, fc1_b, fc2_w_t, fc2_b)


def kernel(x, conv1_w, conv1_b,
           convs_w_0, convs_b_0, convs_w_1, convs_b_1,
           convs_w_2, convs_b_2, convs_w_3, convs_b_3,
           convs_w_4, convs_b_4,
           fc1_w_t, fc1_b, fc2_w_t, fc2_b):
    N, _, H, W = x.shape
    C = conv1_w.shape[1]

    xs = x[:, 0, :, :]
    xp = jnp.pad(xs, ((0, 0), (1, 1), (1, 1 + 8)))
    x9 = jnp.stack([xp[:, dy:dy + H, dx:dx + W + 8]
                    for dy in range(3) for dx in range(3)],
                   axis=-1).astype(jnp.bfloat16)

    convs_w = [convs_w_0, convs_w_1, convs_w_2, convs_w_3, convs_w_4]
    convs_b = [convs_b_0, convs_b_1, convs_b_2, convs_b_3, convs_b_4]
    convs_w = [w.reshape(3, 3, C, C).transpose(1, 2, 0, 3).reshape(3 * C, 3 * C)
               .astype(jnp.bfloat16) for w in convs_w]

    feat = _conv_stack(x9, conv1_w.astype(jnp.bfloat16), conv1_b,
                       convs_w, convs_b)
    x_flat = feat.reshape(N, (H // 2) * (W // 2) * C)
    return _dense_head(x_flat, fc1_w_t, fc1_b, fc2_w_t, fc2_b)


# speedup vs baseline: 1.0805x; 1.0011x over previous
import jax
import jax.numpy as jnp
from jax.experimental import pallas as pl
from jax.experimental.pallas import tpu as pltpu


def _make_conv_stack_kernel(n_hidden, H, W, C, B):
    WW = W + 8
    Hh, Wh = H // 2, W // 2
    M = H * WW
    P3R = (H + 2) * WW

    def body(*refs):
        x_ref = refs[0]
        w1_ref, b1_ref = refs[1], refs[2]
        wb = refs[3:3 + 2 * n_hidden]
        o_ref = refs[3 + 2 * n_hidden]
        p3, obuf, pool_buf = refs[3 + 2 * n_hidden + 1:]

        zb = jnp.zeros((48, 3 * C), jnp.bfloat16)
        p3[pl.ds(0, 48), :] = zb
        p3[pl.ds(P3R - 48, 48), :] = zb

        bidx = jax.lax.broadcasted_iota(jnp.int32, (H, WW, C), 1)
        valid = (bidx < W).reshape(M, C)

        def store_shifted(y):
            yb = jnp.where(valid, y, 0.0).astype(jnp.bfloat16)
            p3[pl.ds(WW + 1, M), 0:C] = yb
            p3[pl.ds(WW, M), C:2 * C] = yb
            p3[pl.ds(WW - 1, M), 2 * C:3 * C] = yb

        for b in range(B):
            y = jax.lax.dot_general(
                x_ref[b].reshape(M, 9), w1_ref[...],
                dimension_numbers=(((1,), (0,)), ((), ())),
                preferred_element_type=jnp.float32)
            store_shifted(jnp.maximum(y + b1_ref[...], 0.0))

            for l in range(n_hidden):
                w_ref, b_ref = wb[2 * l], wb[2 * l + 1]
                obuf[...] = jax.lax.dot_general(
                    p3[...], w_ref[...],
                    dimension_numbers=(((1,), (0,)), ((), ())),
                    preferred_element_type=jnp.float32)
                y = (obuf[pl.ds(0, M), 0:C]
                     + obuf[pl.ds(WW, M), C:2 * C]
                     + obuf[pl.ds(2 * WW, M), 2 * C:3 * C])
                y = jnp.maximum(y + b_ref[...], 0.0)
                if l != n_hidden - 1:
                    store_shifted(y)

            yp = y.reshape(Hh, 2, WW, C)
            pool_buf[...] = jnp.maximum(yp[:, 0], yp[:, 1])
            p = jnp.maximum(pool_buf[:, pl.ds(0, Wh, 2), :],
                            pool_buf[:, pl.ds(1, Wh, 2), :])
            o_ref[b] = p.astype(o_ref.dtype)

    return body


def _conv_stack(x9, conv1_w, conv1_b, convs_w, convs_b):
    N, H, WW, _ = x9.shape
    W = WW - 8
    C = conv1_w.shape[1]
    Hh, Wh = H // 2, W // 2
    n_hidden = len(convs_w)
    B = 2 if N % 2 == 0 else 1

    in_specs = [pl.BlockSpec((B, H, WW, 9), lambda n: (n, 0, 0, 0)),
                pl.BlockSpec((9, C), lambda n: (0, 0)),
                pl.BlockSpec((1, C), lambda n: (0, 0))]
    args = [x9, conv1_w, conv1_b]
    for w, b in zip(convs_w, convs_b):
        in_specs.append(pl.BlockSpec((3 * C, 3 * C), lambda n: (0, 0)))
        in_specs.append(pl.BlockSpec((1, C), lambda n: (0, 0)))
        args.append(w)
        args.append(b)

    body = _make_conv_stack_kernel(n_hidden, H, W, C, B)
    return pl.pallas_call(
        body,
        grid=(N // B,),
        out_shape=jax.ShapeDtypeStruct((N, Hh, Wh, C), jnp.bfloat16),
        in_specs=in_specs,
        out_specs=pl.BlockSpec((B, Hh, Wh, C), lambda n: (n, 0, 0, 0)),
        scratch_shapes=[pltpu.VMEM(((H + 2) * WW, 3 * C), jnp.bfloat16),
                        pltpu.VMEM(((H + 2) * WW, 3 * C), jnp.float32),
                        pltpu.VMEM((Hh, WW, C), jnp.float32)],
        compiler_params=pltpu.CompilerParams(
            dimension_semantics=("parallel",),
            vmem_limit_bytes=64 * 1024 * 1024),
    )(*args)


def _make_dense_head_kernel(nk):
    def body(x_ref, w1_ref, b1_ref, w2_ref, b2_ref, o_ref, acc_ref):
        k = pl.program_id(1)
        part = jax.lax.dot_general(
            x_ref[...], w1_ref[...].astype(jnp.bfloat16),
            dimension_numbers=(((1,), (0,)), ((), ())),
            preferred_element_type=jnp.float32)
        @pl.when(k == 0)
        def _():
            acc_ref[...] = part

        @pl.when(k > 0)
        def _():
            acc_ref[...] += part

        @pl.when(k == nk - 1)
        def _():
            h = jnp.maximum(acc_ref[...] + b1_ref[...], 0.0)
            o_ref[...] = (jax.lax.dot_general(
                h, w2_ref[...],
                dimension_numbers=(((1,), (0,)), ((), ())),
                preferred_element_type=jnp.float32) + b2_ref[...]
            ).astype(o_ref.dtype)

    return body


def _dense_head(x_flat, fc1_w_t, fc1_b, fc2_w_t, fc2_b):
    N, F = x_flat.shape
    hidden = fc1_w_t.shape[1]
    ncls = fc2_w_t.shape[1]
    nb = 1
    Mb = N // nb
    nk, Fb = 4, F // 4
    return pl.pallas_call(
        _make_dense_head_kernel(nk),
        grid=(nb, nk),
        out_shape=jax.ShapeDtypeStruct((N, ncls), jnp.float32),
        in_specs=[
            pl.BlockSpec((Mb, Fb), lambda n, k: (n, k)),
            pl.BlockSpec((Fb, hidden), lambda n, k: (k, 0)),
            pl.BlockSpec((1, hidden), lambda n, k: (0, 0)),
            pl.BlockSpec((hidden, ncls), lambda n, k: (0, 0)),
            pl.BlockSpec((1, ncls), lambda n, k: (0, 0)),
        ],
        out_specs=pl.BlockSpec((Mb, ncls), lambda n, k: (n, 0)),
        scratch_shapes=[pltpu.VMEM((Mb, hidden), jnp.float32)],
        compiler_params=pltpu.CompilerParams(
            dimension_semantics=("parallel", "arbitrary"),
            vmem_limit_bytes=64 * 1024 * 1024),
    )(x_flat, fc1_w_t, fc1_b, fc2_w_t, fc2_b)


def kernel(x, conv1_w, conv1_b,
           convs_w_0, convs_b_0, convs_w_1, convs_b_1,
           convs_w_2, convs_b_2, convs_w_3, convs_b_3,
           convs_w_4, convs_b_4,
           fc1_w_t, fc1_b, fc2_w_t, fc2_b):
    N, _, H, W = x.shape
    C = conv1_w.shape[1]

    xs = x[:, 0, :, :]
    xp = jnp.pad(xs, ((0, 0), (1, 1), (1, 1 + 8)))
    x9 = jnp.stack([xp[:, dy:dy + H, dx:dx + W + 8]
                    for dy in range(3) for dx in range(3)],
                   axis=-1).astype(jnp.bfloat16)

    convs_w = [convs_w_0, convs_w_1, convs_w_2, convs_w_3, convs_w_4]
    convs_b = [convs_b_0, convs_b_1, convs_b_2, convs_b_3, convs_b_4]
    convs_w = [w.reshape(3, 3, C, C).transpose(1, 2, 0, 3).reshape(3 * C, 3 * C)
               .astype(jnp.bfloat16) for w in convs_w]

    feat = _conv_stack(x9, conv1_w.astype(jnp.bfloat16), conv1_b,
                       convs_w, convs_b)
    x_flat = feat.reshape(N, (H // 2) * (W // 2) * C)
    return _dense_head(x_flat, fc1_w_t, fc1_b, fc2_w_t, fc2_b)
